# baseline (speedup 1.0000x reference)
import jax
import jax.numpy as jnp
from jax import lax
from jax.experimental import pallas as pl
from jax.experimental.pallas import tpu as pltpu
from jax.experimental.pallas import tpu_sc as plsc

_NC = 2
_NS = 16
_NW = _NC * _NS
_B = 16384
_H = 50
_D = 64
_BW = _B // _NW
_CH = 128
_NCHUNK = _H * (_BW // _CH)


def _iota16():
    return lax.iota(jnp.int32, 16)


def _gather_body(tableP, idx_t, out_t, idxblk, pidx, parb, rows, outv,
                 gsem, wsem):
    w = lax.axis_index("s") * _NC + lax.axis_index("c")
    b0 = w * _BW
    pltpu.sync_copy(idx_t.at[:, pl.ds(b0, _BW)], idxblk)

    def prep_and_fire(t, s):
        h = t // 4
        cc = t & 3
        hvec = jnp.full((16,), 0, jnp.int32) + h
        for jg in range(8):
            cvec = _CH * cc + 16 * jg + _iota16()
            v = plsc.load_gather(idxblk, [hvec, cvec])
            pidx[s, pl.ds(jg * 16, 16)] = jnp.right_shift(v, 1)
            parb[s, jg, :] = jnp.left_shift(jnp.bitwise_and(v, 1), 6)
        pltpu.async_copy(tableP.at[pidx.at[s]], rows.at[s], gsem.at[s])

    def drain_gathers(s):
        pltpu.make_async_copy(
            tableP.at[pidx.at[s]], rows.at[s], gsem.at[s]).wait()

    def drain_wb():
        pltpu.make_async_copy(
            outv.at[0], out_t.at[0].at[:, pl.ds(0, _CH)], wsem
        ).wait()

    def assemble_and_wb(u, s):
        h = u // 4
        cc = u & 3
        for jg in range(8):
            parv = parb[s, jg, :]
            jvec = 16 * jg + _iota16()

            @plsc.parallel_loop(0, _D, unroll=8)
            def dloop(d):
                colv = parv + d
                vals = plsc.load_gather(rows.at[s], [jvec, colv])
                dvec = jnp.full((16,), 0, jnp.int32) + d
                plsc.store_scatter(outv.at[s], [dvec, jvec], vals)

        pltpu.async_copy(
            outv.at[s],
            out_t.at[h].at[:, pl.ds(b0 + _CH * cc, _CH)],
            wsem,
        )

    def group(g, carry):
        for b in range(2):
            t = 2 * g + b

            @pl.when(t < _NCHUNK)
            def _():
                prep_and_fire(t, b)

            @pl.when((t >= 1) & (t <= _NCHUNK))
            def _():
                drain_gathers(1 - b)

                @pl.when(t >= 3)
                def _():
                    drain_wb()

                assemble_and_wb(t - 1, 1 - b)

        return carry

    lax.fori_loop(0, _NCHUNK // 2 + 1, group, 0)
    drain_wb()
    drain_wb()


@jax.jit
def _gather(tableP, idx_t):
    mesh = plsc.VectorSubcoreMesh(core_axis_name="c", subcore_axis_name="s")
    f = pl.kernel(
        _gather_body,
        mesh=mesh,
        out_type=jax.ShapeDtypeStruct((_H, _D, _B), jnp.float32),
        scratch_types=[
            pltpu.VMEM((_H, _BW), jnp.int32),
            pltpu.VMEM((2, 128), jnp.int32),
            pltpu.VMEM((2, 8, 16), jnp.int32),
            pltpu.VMEM((2, _CH, 128), jnp.float32),
            pltpu.VMEM((2, _D, _CH), jnp.float32),
            pltpu.SemaphoreType.DMA((2,)),
            pltpu.SemaphoreType.DMA,
        ],
        compiler_params=pltpu.CompilerParams(use_tc_tiling_on_sc=True, needs_layout_passes=False),
    )
    return f(tableP, idx_t)


def kernel(input_ids, embedding_table):
    idx_t = input_ids.T
    tableP = embedding_table.reshape(500000, 128)
    out_t = _gather(tableP, idx_t)
    return jnp.transpose(out_t, (2, 0, 1))


# speedup vs baseline: 1.0452x; 1.0452x over previous
import jax
import jax.numpy as jnp
from jax import lax
from jax.experimental import pallas as pl
from jax.experimental.pallas import tpu as pltpu
from jax.experimental.pallas import tpu_sc as plsc

_NC = 2
_NS = 16
_NW = _NC * _NS
_B = 16384
_H = 50
_D = 64
_BW = _B // _NW
_CH = 128
_NCHUNK = _H * (_BW // _CH)


def _iota16():
    return lax.iota(jnp.int32, 16)


def _gather_body(tableP, idx_t, out_t, idxblk, pidx, parb, rows, outv,
                 gsem, wsem):
    w = lax.axis_index("s") * _NC + lax.axis_index("c")
    b0 = w * _BW
    pltpu.sync_copy(idx_t.at[:, pl.ds(b0, _BW)], idxblk)

    def prep_and_fire(t, s):
        h = t // 4
        cc = t & 3
        hvec = jnp.full((16,), 0, jnp.int32) + h
        for jg in range(8):
            cvec = _CH * cc + 16 * jg + _iota16()
            v = plsc.load_gather(idxblk, [hvec, cvec])
            pidx[s, pl.ds(jg * 16, 16)] = jnp.right_shift(v, 1)
            parb[s, jg, :] = jnp.left_shift(jnp.bitwise_and(v, 1), 6)
        pltpu.async_copy(tableP.at[pidx.at[s]], rows.at[s], gsem.at[s])

    def drain_gathers(s):
        pltpu.make_async_copy(
            tableP.at[pidx.at[s]], rows.at[s], gsem.at[s]).wait()

    def drain_wb():
        pltpu.make_async_copy(
            outv.at[0], out_t.at[0].at[:, pl.ds(0, _CH)], wsem
        ).wait()

    def assemble_and_wb(u, s):
        h = u // 4
        cc = u & 3
        for jg in range(8):
            parv = parb[s, jg, :]
            jvec = 16 * jg + _iota16()

            @plsc.parallel_loop(0, _D, unroll=8, carry=parv)
            def dloop(d, colv):
                vals = plsc.load_gather(rows.at[s], [jvec, colv])
                outv[s, d, pl.ds(16 * jg, 16)] = vals
                return colv + 1

        pltpu.async_copy(
            outv.at[s],
            out_t.at[h].at[:, pl.ds(b0 + _CH * cc, _CH)],
            wsem,
        )

    def group(g, carry):
        for b in range(2):
            t = 2 * g + b

            @pl.when(t < _NCHUNK)
            def _():
                prep_and_fire(t, b)

            @pl.when((t >= 1) & (t <= _NCHUNK))
            def _():
                drain_gathers(1 - b)

                @pl.when(t >= 3)
                def _():
                    drain_wb()

                assemble_and_wb(t - 1, 1 - b)

        return carry

    lax.fori_loop(0, _NCHUNK // 2 + 1, group, 0)
    drain_wb()
    drain_wb()


@jax.jit
def _gather(tableP, idx_t):
    mesh = plsc.VectorSubcoreMesh(core_axis_name="c", subcore_axis_name="s")
    f = pl.kernel(
        _gather_body,
        mesh=mesh,
        out_type=jax.ShapeDtypeStruct((_H, _D, _B), jnp.float32),
        scratch_types=[
            pltpu.VMEM((_H, _BW), jnp.int32),
            pltpu.VMEM((2, 128), jnp.int32),
            pltpu.VMEM((2, 8, 16), jnp.int32),
            pltpu.VMEM((2, _CH, 128), jnp.float32),
            pltpu.VMEM((2, _D, _CH), jnp.float32),
            pltpu.SemaphoreType.DMA((2,)),
            pltpu.SemaphoreType.DMA,
        ],
        compiler_params=pltpu.CompilerParams(use_tc_tiling_on_sc=True, needs_layout_passes=False),
    )
    return f(tableP, idx_t)


def kernel(input_ids, embedding_table):
    idx_t = input_ids.T
    tableP = embedding_table.reshape(500000, 128)
    out_t = _gather(tableP, idx_t)
    return jnp.transpose(out_t, (2, 0, 1))


# speedup vs baseline: 1.5470x; 1.4801x over previous
import jax
import jax.numpy as jnp
from jax import lax
from jax.experimental import pallas as pl
from jax.experimental.pallas import tpu as pltpu
from jax.experimental.pallas import tpu_sc as plsc

_NC = 2
_NS = 16
_NW = _NC * _NS
_B = 16384
_H = 50
_D = 64
_BW = _B // _NW
_CH = 128
_NCHUNK = _H * (_BW // _CH)


def _iota16():
    return lax.iota(jnp.int32, 16)


def _gather_body(tableP, idx_t, out_t, idxblk, pidx, parb, rows, outv,
                 gsem, wsem):
    w = lax.axis_index("s") * _NC + lax.axis_index("c")
    b0 = w * _BW
    pltpu.sync_copy(idx_t.at[:, pl.ds(b0, _BW)], idxblk)

    def prep_and_fire(t, s):
        h = t // 4
        cc = t & 3
        hvec = jnp.full((16,), 0, jnp.int32) + h
        for jg in range(8):
            cvec = _CH * cc + 16 * jg + _iota16()
            v = plsc.load_gather(idxblk, [hvec, cvec])
            pidx[s, pl.ds(jg * 16, 16)] = jnp.right_shift(v, 1)
            parb[s, jg, :] = jnp.left_shift(jnp.bitwise_and(v, 1), 6)
        pltpu.async_copy(tableP.at[pidx.at[s]], rows.at[s], gsem.at[s])

    def drain_gathers(s):
        pltpu.make_async_copy(
            tableP.at[pidx.at[s]], rows.at[s], gsem.at[s]).wait()

    def drain_wb():
        pltpu.make_async_copy(
            outv.at[0], out_t.at[0].at[:, pl.ds(0, _CH)], wsem
        ).wait()

    def assemble_and_wb(u, s):
        h = u // 4
        cc = u & 3
        iot = _iota16()
        for jg in range(8):
            parv = parb[s, jg, :]
            jvec = 16 * jg + iot

            @plsc.parallel_loop(0, _D, unroll=8)
            def dloop(i):
                dv = (i & ~15) + ((iot + i) & 15)
                vals = plsc.load_gather(rows.at[s], [jvec, parv + dv])
                plsc.store_scatter(outv.at[s], [dv, jvec], vals)

        pltpu.async_copy(
            outv.at[s],
            out_t.at[h].at[:, pl.ds(b0 + _CH * cc, _CH)],
            wsem,
        )

    def group(g, carry):
        for b in range(2):
            t = 2 * g + b

            @pl.when(t < _NCHUNK)
            def _():
                prep_and_fire(t, b)

            @pl.when((t >= 1) & (t <= _NCHUNK))
            def _():
                drain_gathers(1 - b)

                @pl.when(t >= 3)
                def _():
                    drain_wb()

                assemble_and_wb(t - 1, 1 - b)

        return carry

    lax.fori_loop(0, _NCHUNK // 2 + 1, group, 0)
    drain_wb()
    drain_wb()


@jax.jit
def _gather(tableP, idx_t):
    mesh = plsc.VectorSubcoreMesh(core_axis_name="c", subcore_axis_name="s")
    f = pl.kernel(
        _gather_body,
        mesh=mesh,
        out_type=jax.ShapeDtypeStruct((_H, _D, _B), jnp.float32),
        scratch_types=[
            pltpu.VMEM((_H, _BW), jnp.int32),
            pltpu.VMEM((2, 128), jnp.int32),
            pltpu.VMEM((2, 8, 16), jnp.int32),
            pltpu.VMEM((2, _CH, 128), jnp.float32),
            pltpu.VMEM((2, _D, _CH), jnp.float32),
            pltpu.SemaphoreType.DMA((2,)),
            pltpu.SemaphoreType.DMA,
        ],
        compiler_params=pltpu.CompilerParams(use_tc_tiling_on_sc=True, needs_layout_passes=False),
    )
    return f(tableP, idx_t)


def kernel(input_ids, embedding_table):
    idx_t = input_ids.T
    tableP = embedding_table.reshape(500000, 128)
    out_t = _gather(tableP, idx_t)
    return jnp.transpose(out_t, (2, 0, 1))


# speedup vs baseline: 2.5466x; 1.6462x over previous
import jax
import jax.numpy as jnp
from jax import lax
from jax.experimental import pallas as pl
from jax.experimental.pallas import tpu as pltpu
from jax.experimental.pallas import tpu_sc as plsc

_NC = 2
_NS = 16
_NW = _NC * _NS
_B = 16384
_H = 50
_D = 64
_BW = _B // _NW
_CH = 128
_NCHUNK = _H * (_BW // _CH)


def _iota16():
    return lax.iota(jnp.int32, 16)


def _gather_body(tableP, idx_t, out_t, idxblk, pidx, parb, rows, outv,
                 gsem, wsem):
    w = lax.axis_index("s") * _NC + lax.axis_index("c")
    b0 = w * _BW
    pltpu.sync_copy(idx_t.at[:, pl.ds(b0, _BW)], idxblk)

    def prep_and_fire(t, s):
        h = t // 4
        cc = t & 3
        hvec = jnp.full((16,), 0, jnp.int32) + h
        for jg in range(8):
            cvec = _CH * cc + 16 * jg + _iota16()
            v = plsc.load_gather(idxblk, [hvec, cvec])
            pidx[s, pl.ds(jg * 16, 16)] = jnp.right_shift(v, 1)
            parb[s, jg, :] = jnp.left_shift(jnp.bitwise_and(v, 1), 6)
        pltpu.async_copy(tableP.at[pidx.at[s]], rows.at[s], gsem.at[s])

    def drain_gathers(s):
        pltpu.make_async_copy(
            tableP.at[pidx.at[s]], rows.at[s], gsem.at[s]).wait()

    def drain_wb():
        pltpu.make_async_copy(
            outv.at[0], out_t.at[0].at[:, pl.ds(0, _CH)], wsem
        ).wait()

    def assemble_and_wb(u, s):
        h = u // 4
        cc = u & 3
        iot = _iota16()
        for jg in range(8):
            parv = parb[s, jg, :]
            jvec = 16 * jg + iot

            @plsc.parallel_loop(0, _D, unroll=8)
            def dloop(i):
                dv = (i & ~15) + ((iot + i) & 15)
                vals = plsc.load_gather(rows.at[s], [jvec, parv + dv])
                plsc.store_scatter(outv.at[s], [dv, jvec], vals)

        pltpu.async_copy(
            outv.at[s],
            out_t.at[h].at[:, pl.ds(b0 + _CH * cc, _CH)],
            wsem,
        )

    def group(g, carry):
        for b in range(2):
            t = 2 * g + b

            @pl.when(t < _NCHUNK)
            def _():
                prep_and_fire(t, b)

            @pl.when((t >= 1) & (t <= _NCHUNK))
            def _():
                drain_gathers(1 - b)

                @pl.when(t >= 3)
                def _():
                    drain_wb()

                assemble_and_wb(t - 1, 1 - b)

        return carry

    lax.fori_loop(0, _NCHUNK // 2 + 1, group, 0)
    drain_wb()
    drain_wb()


_VB = 7812


def _transpose_body(table_t, tail_in, tableP, inb, outb, tailb, gsem, wsem):
    w = lax.axis_index("s") * _NC + lax.axis_index("c")
    iot = _iota16()

    def fetch(c, s):
        pltpu.async_copy(
            table_t.at[:, pl.ds(c * 128, 128)], inb.at[s], gsem.at[s]
        )

    def drain_fetch(s):
        pltpu.make_async_copy(
            table_t.at[:, pl.ds(0, 128)], inb.at[s], gsem.at[s]
        ).wait()

    def drain_wb():
        pltpu.make_async_copy(
            outb.at[0], tableP.at[pl.ds(0, 64)], wsem
        ).wait()

    def assemble(c, s):
        for lg in range(8):
            rowv = (16 * lg + iot) & 63
            par = lg // 4
            lvec = 16 * lg + iot

            @plsc.parallel_loop(0, 64, unroll=8)
            def ploop(i):
                pv = (i & ~15) + ((iot + i) & 15)
                vals = plsc.load_gather(inb.at[s], [rowv, 2 * pv + par])
                plsc.store_scatter(outb.at[s], [pv, lvec], vals)

        pltpu.async_copy(outb.at[s], tableP.at[pl.ds(64 * c, 64)], wsem)

    def group(g, carry):
        for b in range(2):
            t = 2 * g + b
            c = w + _NW * t

            @pl.when(c < _VB)
            def _():
                fetch(c, b)

            u = t - 1
            cu = w + _NW * u

            @pl.when((u >= 0) & (cu < _VB))
            def _():
                drain_fetch(1 - b)

                @pl.when(u >= 2)
                def _():
                    drain_wb()

                assemble(cu, 1 - b)

        return carry

    lax.fori_loop(0, (_VB // _NW + 2) // 2 + 1, group, 0)
    drain_wb()
    drain_wb()

    @pl.when(w == _NW - 1)
    def _():
        pltpu.sync_copy(tail_in, tailb)
        pltpu.sync_copy(tailb, tableP.at[pl.ds(_VB * 64, 32)])


@jax.jit
def _transpose(table_t, tail_in):
    mesh = plsc.VectorSubcoreMesh(core_axis_name="c", subcore_axis_name="s")
    f = pl.kernel(
        _transpose_body,
        mesh=mesh,
        out_type=jax.ShapeDtypeStruct((_VB * 64 + 32, 128), jnp.float32),
        scratch_types=[
            pltpu.VMEM((2, 64, 128), jnp.float32),
            pltpu.VMEM((2, 64, 128), jnp.float32),
            pltpu.VMEM((32, 128), jnp.float32),
            pltpu.SemaphoreType.DMA((2,)),
            pltpu.SemaphoreType.DMA,
        ],
        compiler_params=pltpu.CompilerParams(
            use_tc_tiling_on_sc=True, needs_layout_passes=False),
    )
    return f(table_t, tail_in)


@jax.jit
def _gather(tableP, idx_t):
    mesh = plsc.VectorSubcoreMesh(core_axis_name="c", subcore_axis_name="s")
    f = pl.kernel(
        _gather_body,
        mesh=mesh,
        out_type=jax.ShapeDtypeStruct((_H, _D, _B), jnp.float32),
        scratch_types=[
            pltpu.VMEM((_H, _BW), jnp.int32),
            pltpu.VMEM((2, 128), jnp.int32),
            pltpu.VMEM((2, 8, 16), jnp.int32),
            pltpu.VMEM((2, _CH, 128), jnp.float32),
            pltpu.VMEM((2, _D, _CH), jnp.float32),
            pltpu.SemaphoreType.DMA((2,)),
            pltpu.SemaphoreType.DMA,
        ],
        compiler_params=pltpu.CompilerParams(use_tc_tiling_on_sc=True, needs_layout_passes=False),
    )
    return f(tableP, idx_t)


def kernel(input_ids, embedding_table):
    idx_t = input_ids.T
    table_t = embedding_table.T
    tail_in = embedding_table[_VB * 128:].reshape(32, 128)
    tableP = _transpose(table_t, tail_in)
    out_t = _gather(tableP, idx_t)
    return jnp.transpose(out_t, (2, 0, 1))


# speedup vs baseline: 2.6921x; 1.0571x over previous
import jax
import jax.numpy as jnp
from jax import lax
from jax.experimental import pallas as pl
from jax.experimental.pallas import tpu as pltpu
from jax.experimental.pallas import tpu_sc as plsc

_NC = 2
_NS = 16
_NW = _NC * _NS
_B = 16384
_H = 50
_D = 64
_BW = _B // _NW
_CH = 128
_NCHUNK = _H * (_BW // _CH)


def _iota16():
    return lax.iota(jnp.int32, 16)


def _gather_body(tableP, idx_t, out_t, idxblk, pidx, parb, rows, outv,
                 gsem, wsem):
    w = lax.axis_index("s") * _NC + lax.axis_index("c")
    b0 = w * _BW
    pltpu.sync_copy(idx_t.at[:, pl.ds(b0, _BW)], idxblk)

    def prep_and_fire(t, s):
        h = t // 4
        cc = t & 3
        hvec = jnp.full((16,), 0, jnp.int32) + h
        for jg in range(8):
            cvec = _CH * cc + 16 * jg + _iota16()
            v = plsc.load_gather(idxblk, [hvec, cvec])
            pidx[s, pl.ds(jg * 16, 16)] = jnp.right_shift(v, 1)
            parb[s, jg, :] = jnp.left_shift(jnp.bitwise_and(v, 1), 6)
        pltpu.async_copy(tableP.at[pidx.at[s]], rows.at[s], gsem.at[s])

    def drain_gathers(s):
        pltpu.make_async_copy(
            tableP.at[pidx.at[s]], rows.at[s], gsem.at[s]).wait()

    def drain_wb():
        pltpu.make_async_copy(
            outv.at[0], out_t.at[0].at[:, pl.ds(0, _CH)], wsem
        ).wait()

    def assemble_and_wb(u, s):
        h = u // 4
        cc = u & 3
        iot = _iota16()
        for jg in range(8):
            parv = parb[s, jg, :]
            jvec = 16 * jg + iot

            @plsc.parallel_loop(0, _D, unroll=16)
            def dloop(i):
                dv = (i & ~15) + ((iot + i) & 15)
                vals = plsc.load_gather(rows.at[s], [jvec, parv + dv])
                plsc.store_scatter(outv.at[s], [dv, jvec], vals)

        pltpu.async_copy(
            outv.at[s],
            out_t.at[h].at[:, pl.ds(b0 + _CH * cc, _CH)],
            wsem,
        )

    def group(g, carry):
        for b in range(2):
            t = 2 * g + b

            @pl.when(t < _NCHUNK)
            def _():
                prep_and_fire(t, b)

            @pl.when((t >= 1) & (t <= _NCHUNK))
            def _():
                drain_gathers(1 - b)

                @pl.when(t >= 3)
                def _():
                    drain_wb()

                assemble_and_wb(t - 1, 1 - b)

        return carry

    lax.fori_loop(0, _NCHUNK // 2 + 1, group, 0)
    drain_wb()
    drain_wb()


_VB = 7812


def _transpose_body(table_t, tail_in, tableP, inb, outb, tailb, gsem, wsem):
    w = lax.axis_index("s") * _NC + lax.axis_index("c")
    iot = _iota16()

    def fetch(c, s):
        pltpu.async_copy(
            table_t.at[:, pl.ds(c * 128, 128)], inb.at[s], gsem.at[s]
        )

    def drain_fetch(s):
        pltpu.make_async_copy(
            table_t.at[:, pl.ds(0, 128)], inb.at[s], gsem.at[s]
        ).wait()

    def drain_wb():
        pltpu.make_async_copy(
            outb.at[0], tableP.at[pl.ds(0, 64)], wsem
        ).wait()

    def assemble(c, s):
        for lg in range(8):
            rowv = (16 * lg + iot) & 63
            par = lg // 4
            lvec = 16 * lg + iot

            @plsc.parallel_loop(0, 64, unroll=16)
            def ploop(i):
                pv = (i & ~15) + ((iot + i) & 15)
                vals = plsc.load_gather(inb.at[s], [rowv, 2 * pv + par])
                plsc.store_scatter(outb.at[s], [pv, lvec], vals)

        pltpu.async_copy(outb.at[s], tableP.at[pl.ds(64 * c, 64)], wsem)

    def group(g, carry):
        for b in range(2):
            t = 2 * g + b
            c = w + _NW * t

            @pl.when(c < _VB)
            def _():
                fetch(c, b)

            u = t - 1
            cu = w + _NW * u

            @pl.when((u >= 0) & (cu < _VB))
            def _():
                drain_fetch(1 - b)

                @pl.when(u >= 2)
                def _():
                    drain_wb()

                assemble(cu, 1 - b)

        return carry

    lax.fori_loop(0, (_VB // _NW + 2) // 2 + 1, group, 0)
    drain_wb()
    drain_wb()

    @pl.when(w == _NW - 1)
    def _():
        pltpu.sync_copy(tail_in, tailb)
        pltpu.sync_copy(tailb, tableP.at[pl.ds(_VB * 64, 32)])


@jax.jit
def _transpose(table_t, tail_in):
    mesh = plsc.VectorSubcoreMesh(core_axis_name="c", subcore_axis_name="s")
    f = pl.kernel(
        _transpose_body,
        mesh=mesh,
        out_type=jax.ShapeDtypeStruct((_VB * 64 + 32, 128), jnp.float32),
        scratch_types=[
            pltpu.VMEM((2, 64, 128), jnp.float32),
            pltpu.VMEM((2, 64, 128), jnp.float32),
            pltpu.VMEM((32, 128), jnp.float32),
            pltpu.SemaphoreType.DMA((2,)),
            pltpu.SemaphoreType.DMA,
        ],
        compiler_params=pltpu.CompilerParams(
            use_tc_tiling_on_sc=True, needs_layout_passes=False),
    )
    return f(table_t, tail_in)


@jax.jit
def _gather(tableP, idx_t):
    mesh = plsc.VectorSubcoreMesh(core_axis_name="c", subcore_axis_name="s")
    f = pl.kernel(
        _gather_body,
        mesh=mesh,
        out_type=jax.ShapeDtypeStruct((_H, _D, _B), jnp.float32),
        scratch_types=[
            pltpu.VMEM((_H, _BW), jnp.int32),
            pltpu.VMEM((2, 128), jnp.int32),
            pltpu.VMEM((2, 8, 16), jnp.int32),
            pltpu.VMEM((2, _CH, 128), jnp.float32),
            pltpu.VMEM((2, _D, _CH), jnp.float32),
            pltpu.SemaphoreType.DMA((2,)),
            pltpu.SemaphoreType.DMA,
        ],
        compiler_params=pltpu.CompilerParams(use_tc_tiling_on_sc=True, needs_layout_passes=False),
    )
    return f(tableP, idx_t)


def kernel(input_ids, embedding_table):
    idx_t = input_ids.T
    table_t = embedding_table.T
    tail_in = embedding_table[_VB * 128:].reshape(32, 128)
    tableP = _transpose(table_t, tail_in)
    out_t = _gather(tableP, idx_t)
    return jnp.transpose(out_t, (2, 0, 1))


# speedup vs baseline: 2.8831x; 1.0709x over previous
import jax
import jax.numpy as jnp
from jax import lax
from jax.experimental import pallas as pl
from jax.experimental.pallas import tpu as pltpu
from jax.experimental.pallas import tpu_sc as plsc

_NC = 2
_NS = 16
_NW = _NC * _NS
_B = 16384
_H = 50
_D = 64
_BW = _B // _NW
_CH = 128
_NCHUNK = _H * (_BW // _CH)


def _iota16():
    return lax.iota(jnp.int32, 16)


def _gather_body(tableP, idx_t, out_t, idxblk, pidx, parb, rows, outv,
                 gsem, wsem):
    w = lax.axis_index("s") * _NC + lax.axis_index("c")
    b0 = w * _BW
    pltpu.sync_copy(idx_t.at[:, pl.ds(b0, _BW)], idxblk)

    def prep_and_fire(t, s):
        h = t // 4
        cc = t & 3
        hvec = jnp.full((16,), 0, jnp.int32) + h
        for jg in range(8):
            cvec = _CH * cc + 16 * jg + _iota16()
            v = plsc.load_gather(idxblk, [hvec, cvec])
            pidx[s, pl.ds(jg * 16, 16)] = jnp.right_shift(v, 1)
            parb[s, jg, :] = jnp.left_shift(jnp.bitwise_and(v, 1), 6)
        pltpu.async_copy(tableP.at[pidx.at[s]], rows.at[s], gsem.at[s])

    def drain_gathers(s):
        pltpu.make_async_copy(
            tableP.at[pidx.at[s]], rows.at[s], gsem.at[s]).wait()

    def drain_wb():
        pltpu.make_async_copy(
            outv.at[0], out_t.at[0].at[:, pl.ds(0, _CH)], wsem
        ).wait()

    def assemble_and_wb(u, s):
        h = u // 4
        cc = u & 3
        iot = _iota16()
        for jg in range(8):
            parv = parb[s, jg, :]
            jvec = 16 * jg + iot

            @plsc.parallel_loop(0, _D, unroll=16)
            def dloop(i):
                dv = (i & ~15) + ((iot + i) & 15)
                vals = plsc.load_gather(rows.at[s], [jvec, parv + dv])
                plsc.store_scatter(outv.at[s], [dv, jvec], vals)

        pltpu.async_copy(
            outv.at[s],
            out_t.at[h].at[:, pl.ds(b0 + _CH * cc, _CH)],
            wsem,
        )

    def group(g, carry):
        for b in range(2):
            t = 2 * g + b

            @pl.when(t < _NCHUNK)
            def _():
                prep_and_fire(t, b)

            @pl.when((t >= 1) & (t <= _NCHUNK))
            def _():
                drain_gathers(1 - b)

                @pl.when(t >= 3)
                def _():
                    drain_wb()

                assemble_and_wb(t - 1, 1 - b)

        return carry

    lax.fori_loop(0, _NCHUNK // 2 + 1, group, 0)
    drain_wb()
    drain_wb()


_VB2 = 3906


def _transpose_body(table_t, tail_in, tableP, inb, outb, tailb, gsem, wsem):
    w = lax.axis_index("s") * _NC + lax.axis_index("c")
    iot = _iota16()

    def fetch(c, s):
        pltpu.async_copy(
            table_t.at[:, pl.ds(c * 256, 256)], inb.at[s], gsem.at[s]
        )

    def drain_fetch(s):
        pltpu.make_async_copy(
            table_t.at[:, pl.ds(0, 256)], inb.at[s], gsem.at[s]
        ).wait()

    def drain_wb():
        pltpu.make_async_copy(
            outb.at[0], tableP.at[pl.ds(0, 128)], wsem
        ).wait()

    def assemble(c, s):
        for lg in range(8):
            rowv = (16 * lg + iot) & 63
            par = lg // 4
            lvec = 16 * lg + iot

            @plsc.parallel_loop(0, 128, unroll=16)
            def ploop(i):
                pv = (i & ~15) + ((iot + i) & 15)
                vals = plsc.load_gather(inb.at[s], [rowv, 2 * pv + par])
                plsc.store_scatter(outb.at[s], [pv, lvec], vals)

        pltpu.async_copy(outb.at[s], tableP.at[pl.ds(128 * c, 128)], wsem)

    def group(g, carry):
        for b in range(2):
            t = 2 * g + b
            c = w + _NW * t

            @pl.when(c < _VB2)
            def _():
                fetch(c, b)

            u = t - 1
            cu = w + _NW * u

            @pl.when((u >= 0) & (cu < _VB2))
            def _():
                drain_fetch(1 - b)

                @pl.when(u >= 2)
                def _():
                    drain_wb()

                assemble(cu, 1 - b)

        return carry

    lax.fori_loop(0, (_VB2 // _NW + 2) // 2 + 1, group, 0)
    drain_wb()
    drain_wb()

    @pl.when(w == _NW - 1)
    def _():
        pltpu.sync_copy(tail_in, tailb)
        pltpu.sync_copy(tailb, tableP.at[pl.ds(_VB2 * 128, 32)])


@jax.jit
def _transpose(table_t, tail_in):
    mesh = plsc.VectorSubcoreMesh(core_axis_name="c", subcore_axis_name="s")
    f = pl.kernel(
        _transpose_body,
        mesh=mesh,
        out_type=jax.ShapeDtypeStruct((_VB2 * 128 + 32, 128), jnp.float32),
        scratch_types=[
            pltpu.VMEM((2, 64, 256), jnp.float32),
            pltpu.VMEM((2, 128, 128), jnp.float32),
            pltpu.VMEM((32, 128), jnp.float32),
            pltpu.SemaphoreType.DMA((2,)),
            pltpu.SemaphoreType.DMA,
        ],
        compiler_params=pltpu.CompilerParams(
            use_tc_tiling_on_sc=True, needs_layout_passes=False),
    )
    return f(table_t, tail_in)


@jax.jit
def _gather(tableP, idx_t):
    mesh = plsc.VectorSubcoreMesh(core_axis_name="c", subcore_axis_name="s")
    f = pl.kernel(
        _gather_body,
        mesh=mesh,
        out_type=jax.ShapeDtypeStruct((_H, _D, _B), jnp.float32),
        scratch_types=[
            pltpu.VMEM((_H, _BW), jnp.int32),
            pltpu.VMEM((2, 128), jnp.int32),
            pltpu.VMEM((2, 8, 16), jnp.int32),
            pltpu.VMEM((2, _CH, 128), jnp.float32),
            pltpu.VMEM((2, _D, _CH), jnp.float32),
            pltpu.SemaphoreType.DMA((2,)),
            pltpu.SemaphoreType.DMA,
        ],
        compiler_params=pltpu.CompilerParams(use_tc_tiling_on_sc=True, needs_layout_passes=False),
    )
    return f(tableP, idx_t)


def kernel(input_ids, embedding_table):
    idx_t = input_ids.T
    table_t = embedding_table.T
    tail_in = embedding_table[_VB2 * 256:].reshape(32, 128)
    tableP = _transpose(table_t, tail_in)
    out_t = _gather(tableP, idx_t)
    return jnp.transpose(out_t, (2, 0, 1))
